# Initial kernel scaffold; baseline (speedup 1.0000x reference)
import jax
import jax.numpy as jnp
from jax.experimental import pallas as pl


def kernel(x, classifier_weight):
    raise NotImplementedError("write your pallas kernel here")


# speedup vs baseline: 1.1670x; 1.1670x over previous
import functools
import jax
import jax.numpy as jnp
from jax.experimental import pallas as pl
from jax.experimental.pallas import tpu as pltpu

C = 16
RELAX_DENOM = 2.0
NUM_OFF = C * (C - 1) / 2.0
MARGIN = float(int(NUM_OFF // RELAX_DENOM))


def _x_imap(i, b, g, chan_ref, wgh_ref):
    return (b, chan_ref[g * C + i], 0, 0)


def _saw_body(chan_ref, wgh_ref, *refs):
    xrefs = refs[:C]
    out_ref = refs[C]
    g = pl.program_id(1)
    base = g * C
    a = [xrefs[c][0, 0] * wgh_ref[base + c] for c in range(C)]
    parts = []
    for c in range(C):
        for d in range(c + 1, C):
            parts.append(jnp.abs(jnp.sum(a[c] * a[d])))
    tot = functools.reduce(lambda u, v: u + v, parts)
    hw = a[0].shape[0] * a[0].shape[1]
    loss = jnp.maximum((tot / (hw - 1) - MARGIN) / NUM_OFF, 0.0)
    out_ref[0, 0] = jnp.broadcast_to(loss, out_ref.shape[2:])


def kernel(x, classifier_weight):
    B, ch, H, W = x.shape
    G = ch // C
    w = jnp.abs(classifier_weight)
    idx = jnp.argsort(-w, axis=1)
    idx_sel = idx[:C, :G]
    sig = jax.nn.sigmoid(w)[:C]
    chan = idx_sel.T.reshape(-1).astype(jnp.int32)
    wgh = jnp.take_along_axis(sig, idx_sel, axis=1)
    wghf = wgh.T.reshape(-1).astype(jnp.float32)

    x4 = x.reshape(B, ch, (H * W) // 256, 256)

    in_specs = [
        pl.BlockSpec((1, 1, x4.shape[2], 256), functools.partial(_x_imap, i))
        for i in range(C)
    ]
    out = pl.pallas_call(
        _saw_body,
        grid_spec=pltpu.PrefetchScalarGridSpec(
            num_scalar_prefetch=2,
            grid=(B, G),
            in_specs=in_specs,
            out_specs=pl.BlockSpec((1, 1, 8, 128), lambda b, g, c_r, w_r: (b, g, 0, 0)),
        ),
        out_shape=jax.ShapeDtypeStruct((B, G, 8, 128), jnp.float32),
        compiler_params=pltpu.CompilerParams(
            dimension_semantics=("parallel", "arbitrary"),
        ),
        name="saw_loss",
    )(chan, wghf, *([x4] * C))
    total = jnp.sum(out[:, :, 0, 0]) / B
    return total.reshape(1)


# speedup vs baseline: 3.4682x; 2.9719x over previous
import functools
import jax
import jax.numpy as jnp
from jax.experimental import pallas as pl
from jax.experimental.pallas import tpu as pltpu

C = 16
RELAX_DENOM = 2.0
NUM_OFF = C * (C - 1) / 2.0
MARGIN = float(int(NUM_OFF // RELAX_DENOM))

NCH = 32
MROW = 256
KSTEPS = MROW // NCH
SSTR = NCH + 1


def _x_imap(i, b, s, k, chan_ref):
    return (b, chan_ref[s * MROW + k * NCH + i], 0, 0)


def _saw_body(chan_ref, *refs):
    xrefs = refs[:NCH]
    mask_ref = refs[NCH]
    out_ref = refs[NCH + 1]
    tile = refs[NCH + 2]
    k = pl.program_id(2)
    nh = xrefs[0].shape[2]
    tk = tile.at[k]
    for i in range(NCH):
        tk[i : i + SSTR * nh : SSTR, :] = xrefs[i][0, 0]

    @pl.when(k == KSTEPS - 1)
    def _():
        covs = []
        for j in range(nh // 2):
            lhs = jnp.concatenate(
                [
                    jnp.concatenate(
                        [tile[kk, jj * SSTR : jj * SSTR + NCH, :]
                         for kk in range(KSTEPS)],
                        axis=0,
                    )
                    for jj in (2 * j, 2 * j + 1)
                ],
                axis=1,
            )
            covs.append(
                jax.lax.dot_general(
                    lhs, lhs.T, (((1,), (0,)), ((), ())),
                    preferred_element_type=jnp.float32,
                )
            )
        while len(covs) > 1:
            covs = [a + b for a, b in zip(covs[::2], covs[1::2])] + (
                [covs[-1]] if len(covs) % 2 else [])
        t = jnp.abs(covs[0]) * mask_ref[0]
        rs = jnp.sum(t, axis=1, keepdims=True)
        gs = jnp.sum(rs.reshape(C, C, 1), axis=1)
        out_ref[0, 0] = jnp.maximum((gs - MARGIN) / NUM_OFF, 0.0)


def kernel(x, classifier_weight):
    B, ch, H, W = x.shape
    G = ch // C
    nstack = ch // MROW
    hw = H * W
    w = jnp.abs(classifier_weight)
    idx = jnp.argsort(-w, axis=1)
    idx_sel = idx[:C, :G]
    sig = jax.nn.sigmoid(w)[:C]
    chan = idx_sel.T.reshape(-1).astype(jnp.int32)
    wgh = jnp.take_along_axis(sig, idx_sel, axis=1)
    wv = wgh.T.reshape(-1).astype(jnp.float32)

    q = jnp.arange(MROW)
    samegrp = (q[:, None] // C) == (q[None, :] // C)
    upper = q[:, None] < q[None, :]
    bmask = (samegrp & upper).astype(jnp.float32) / (hw - 1)
    ws = wv.reshape(nstack, MROW)
    mask_w = ws[:, :, None] * ws[:, None, :] * bmask[None]

    x4 = x.reshape(B, ch, hw // 128, 128)
    nh = x4.shape[2]

    in_specs = [
        pl.BlockSpec((1, 1, nh, 128), functools.partial(_x_imap, i))
        for i in range(NCH)
    ] + [pl.BlockSpec((1, MROW, MROW), lambda b, s, k, c_r: (s, 0, 0))]

    out = pl.pallas_call(
        _saw_body,
        grid_spec=pltpu.PrefetchScalarGridSpec(
            num_scalar_prefetch=1,
            grid=(B, nstack, KSTEPS),
            in_specs=in_specs,
            out_specs=pl.BlockSpec((1, 1, C, 1), lambda b, s, k, c_r: (b, s, 0, 0)),
            scratch_shapes=[pltpu.VMEM((KSTEPS, SSTR * nh, 128), jnp.float32)],
        ),
        out_shape=jax.ShapeDtypeStruct((B, nstack, C, 1), jnp.float32),
        compiler_params=pltpu.CompilerParams(
            dimension_semantics=("parallel", "arbitrary", "arbitrary"),
        ),
        name="saw_loss_mxu",
    )(chan, *([x4] * NCH), mask_w)
    total = jnp.sum(out) / B
    return total.reshape(1)


# speedup vs baseline: 4.1312x; 1.1912x over previous
import functools
import jax
import jax.numpy as jnp
from jax.experimental import pallas as pl
from jax.experimental.pallas import tpu as pltpu

C = 16
RELAX_DENOM = 2.0
NUM_OFF = C * (C - 1) / 2.0
MARGIN = float(int(NUM_OFF // RELAX_DENOM))

NCH = 64
MROW = 256
KSTEPS = MROW // NCH
SSTR = NCH + 1


def _x_imap(i, b, s, k, chan_ref):
    return (b, chan_ref[s * MROW + k * NCH + i], 0, 0)


def _saw_body(chan_ref, *refs):
    xrefs = refs[:NCH]
    mask_ref = refs[NCH]
    out_ref = refs[NCH + 1]
    tile = refs[NCH + 2]
    k = pl.program_id(2)
    nh = xrefs[0].shape[2]
    tk = tile.at[k]
    for i in range(NCH):
        tk[i : i + SSTR * nh : SSTR, :] = xrefs[i][0, 0]

    @pl.when(k == KSTEPS - 1)
    def _():
        covs = []
        for j in range(nh // 2):
            lhs = jnp.concatenate(
                [
                    jnp.concatenate(
                        [tile[kk, jj * SSTR : jj * SSTR + NCH, :]
                         for kk in range(KSTEPS)],
                        axis=0,
                    )
                    for jj in (2 * j, 2 * j + 1)
                ],
                axis=1,
            )
            covs.append(
                jax.lax.dot_general(
                    lhs, lhs.T, (((1,), (0,)), ((), ())),
                    preferred_element_type=jnp.float32,
                )
            )
        while len(covs) > 1:
            covs = [a + b for a, b in zip(covs[::2], covs[1::2])] + (
                [covs[-1]] if len(covs) % 2 else [])
        t = jnp.abs(covs[0]) * mask_ref[0]
        rs = jnp.sum(t, axis=1, keepdims=True)
        gs = jnp.sum(rs.reshape(C, C, 1), axis=1)
        out_ref[0, 0] = jnp.maximum((gs - MARGIN) / NUM_OFF, 0.0)


def kernel(x, classifier_weight):
    B, ch, H, W = x.shape
    G = ch // C
    nstack = ch // MROW
    hw = H * W
    w = jnp.abs(classifier_weight)
    idx = jnp.argsort(-w, axis=1)
    idx_sel = idx[:C, :G]
    sig = jax.nn.sigmoid(w)[:C]
    chan = idx_sel.T.reshape(-1).astype(jnp.int32)
    wgh = jnp.take_along_axis(sig, idx_sel, axis=1)
    wv = wgh.T.reshape(-1).astype(jnp.float32)

    q = jnp.arange(MROW)
    samegrp = (q[:, None] // C) == (q[None, :] // C)
    upper = q[:, None] < q[None, :]
    bmask = (samegrp & upper).astype(jnp.float32) / (hw - 1)
    ws = wv.reshape(nstack, MROW)
    mask_w = ws[:, :, None] * ws[:, None, :] * bmask[None]

    x4 = x.reshape(B, ch, hw // 128, 128)
    nh = x4.shape[2]

    in_specs = [
        pl.BlockSpec((1, 1, nh, 128), functools.partial(_x_imap, i))
        for i in range(NCH)
    ] + [pl.BlockSpec((1, MROW, MROW), lambda b, s, k, c_r: (s, 0, 0))]

    out = pl.pallas_call(
        _saw_body,
        grid_spec=pltpu.PrefetchScalarGridSpec(
            num_scalar_prefetch=1,
            grid=(B, nstack, KSTEPS),
            in_specs=in_specs,
            out_specs=pl.BlockSpec((1, 1, C, 1), lambda b, s, k, c_r: (b, s, 0, 0)),
            scratch_shapes=[pltpu.VMEM((KSTEPS, SSTR * nh, 128), jnp.float32)],
        ),
        out_shape=jax.ShapeDtypeStruct((B, nstack, C, 1), jnp.float32),
        compiler_params=pltpu.CompilerParams(
            dimension_semantics=("parallel", "arbitrary", "arbitrary"),
        ),
        name="saw_loss_mxu",
    )(chan, *([x4] * NCH), mask_w)
    total = jnp.sum(out) / B
    return total.reshape(1)


# speedup vs baseline: 4.1652x; 1.0082x over previous
import functools
import jax
import jax.numpy as jnp
from jax.experimental import pallas as pl
from jax.experimental.pallas import tpu as pltpu

C = 16
RELAX_DENOM = 2.0
NUM_OFF = C * (C - 1) / 2.0
MARGIN = float(int(NUM_OFF // RELAX_DENOM))

NCH = 128
MROW = 256
KSTEPS = MROW // NCH
SSTR = NCH + 1


def _x_imap(i, b, s, k, chan_ref):
    return (b, chan_ref[s * MROW + k * NCH + i], 0, 0)


def _saw_body(chan_ref, *refs):
    xrefs = refs[:NCH]
    mask_ref = refs[NCH]
    out_ref = refs[NCH + 1]
    tile = refs[NCH + 2]
    k = pl.program_id(2)
    nh = xrefs[0].shape[2]
    tk = tile.at[k]
    for i in range(NCH):
        tk[i : i + SSTR * nh : SSTR, :] = xrefs[i][0, 0]

    @pl.when(k == KSTEPS - 1)
    def _():
        covs = []
        for j in range(nh // 2):
            lhs = jnp.concatenate(
                [
                    jnp.concatenate(
                        [tile[kk, jj * SSTR : jj * SSTR + NCH, :]
                         for kk in range(KSTEPS)],
                        axis=0,
                    )
                    for jj in (2 * j, 2 * j + 1)
                ],
                axis=1,
            )
            covs.append(
                jax.lax.dot_general(
                    lhs, lhs.T, (((1,), (0,)), ((), ())),
                    preferred_element_type=jnp.float32,
                )
            )
        while len(covs) > 1:
            covs = [a + b for a, b in zip(covs[::2], covs[1::2])] + (
                [covs[-1]] if len(covs) % 2 else [])
        t = jnp.abs(covs[0]) * mask_ref[0]
        rs = jnp.sum(t, axis=1, keepdims=True)
        gs = jnp.sum(rs.reshape(C, C, 1), axis=1)
        out_ref[0, 0] = jnp.maximum((gs - MARGIN) / NUM_OFF, 0.0)


def kernel(x, classifier_weight):
    B, ch, H, W = x.shape
    G = ch // C
    nstack = ch // MROW
    hw = H * W
    w = jnp.abs(classifier_weight)
    idx = jnp.argsort(-w, axis=1)
    idx_sel = idx[:C, :G]
    sig = jax.nn.sigmoid(w)[:C]
    chan = idx_sel.T.reshape(-1).astype(jnp.int32)
    wgh = jnp.take_along_axis(sig, idx_sel, axis=1)
    wv = wgh.T.reshape(-1).astype(jnp.float32)

    q = jnp.arange(MROW)
    samegrp = (q[:, None] // C) == (q[None, :] // C)
    upper = q[:, None] < q[None, :]
    bmask = (samegrp & upper).astype(jnp.float32) / (hw - 1)
    ws = wv.reshape(nstack, MROW)
    mask_w = ws[:, :, None] * ws[:, None, :] * bmask[None]

    x4 = x.reshape(B, ch, hw // 128, 128)
    nh = x4.shape[2]

    in_specs = [
        pl.BlockSpec((1, 1, nh, 128), functools.partial(_x_imap, i))
        for i in range(NCH)
    ] + [pl.BlockSpec((1, MROW, MROW), lambda b, s, k, c_r: (s, 0, 0))]

    out = pl.pallas_call(
        _saw_body,
        grid_spec=pltpu.PrefetchScalarGridSpec(
            num_scalar_prefetch=1,
            grid=(B, nstack, KSTEPS),
            in_specs=in_specs,
            out_specs=pl.BlockSpec((1, 1, C, 1), lambda b, s, k, c_r: (b, s, 0, 0)),
            scratch_shapes=[pltpu.VMEM((KSTEPS, SSTR * nh, 128), jnp.float32)],
        ),
        out_shape=jax.ShapeDtypeStruct((B, nstack, C, 1), jnp.float32),
        compiler_params=pltpu.CompilerParams(
            dimension_semantics=("parallel", "arbitrary", "arbitrary"),
        ),
        name="saw_loss_mxu",
    )(chan, *([x4] * NCH), mask_w)
    total = jnp.sum(out) / B
    return total.reshape(1)
